# baseline (speedup 1.0000x reference)
import functools

import jax
import jax.numpy as jnp
from jax import lax
from jax.experimental import pallas as pl
from jax.experimental.pallas import tpu as pltpu
from jax.experimental.pallas import tpu_sc as plsc

_TEMPERATURE = 0.8
_INV_T = 1.0 / _TEMPERATURE
_TOP_K = 50
_TOP_P = 0.95
_FILTER_VALUE = -1e9

_N_ROWS = 128
_N_COLS = 100000
_NV = _N_COLS // 16
_K1 = 25
_S1 = _NV // _K1
_K2 = 25
_S2 = _S1 // _K2
_CAP = 512
_NCV = _CAP // 16
_NCH = 5
_GPC = _S1 // _NCH
_CHE = (_NV // _NCH) * 16
_NEG = -1e30
_LN2 = 0.6931471805599453


def _bitonic_fix(vs):
  if len(vs) == 1:
    return [jnp.sort(vs[0])]
  h = len(vs) // 2
  lo = [jnp.minimum(vs[j], vs[j + h]) for j in range(h)]
  hi = [jnp.maximum(vs[j], vs[j + h]) for j in range(h)]
  return _bitonic_fix(lo) + _bitonic_fix(hi)


def _merge(a, b):
  rb = [lax.rev(x, (0,)) for x in reversed(b)]
  lo = [jnp.minimum(x, y) for x, y in zip(a, rb)]
  hi = [jnp.maximum(x, y) for x, y in zip(a, rb)]
  return _bitonic_fix(lo) + _bitonic_fix(hi)


def _log(x):
  b = plsc.bitcast(x, jnp.int32)
  e2 = (lax.shift_right_arithmetic(b, 23) - 127).astype(jnp.float32)
  m = plsc.bitcast((b & 0x007FFFFF) | 0x3F800000, jnp.float32)
  t = (m - 1.0) / (m + 1.0)
  t2 = t * t
  p = 1.0 + t2 * (1 / 3 + t2 * (1 / 5 + t2 * (1 / 7 + t2 * (1 / 9))))
  return e2 * _LN2 + 2.0 * t * p


def _row_body(row_id, x_hbm, out_hbm, row, sm, sm2s, hotidx, cand, stage,
              sem_in, sem_out):
  iota = lax.iota(jnp.int32, 16)
  x_row = x_hbm.at[row_id]
  out_row = out_hbm.at[row_id]

  pltpu.async_copy(x_row.at[pl.ds(0, _CHE)], row.at[pl.ds(0, _CHE)], sem_in)

  def p1c(c, mcarry):
    @pl.when(c + 1 < _NCH)
    def _():
      off = (c + 1) * _CHE
      pltpu.async_copy(x_row.at[pl.ds(off, _CHE)], row.at[pl.ds(off, _CHE)],
                       sem_in)

    pltpu.make_async_copy(x_row.at[pl.ds(c * _CHE, _CHE)],
                          row.at[pl.ds(c * _CHE, _CHE)], sem_in).wait()

    def p1(s, mc):
      base = c * _CHE + s * (_K1 * 16)
      acc = row[pl.ds(base, 16)]
      for k in range(1, _K1):
        acc = jnp.maximum(acc, row[pl.ds(base + k * 16, 16)])
      sm[pl.ds((c * _GPC + s) * 16, 16)] = acc
      return jnp.maximum(mc, acc)

    return lax.fori_loop(0, _GPC, p1, mcarry)

  mvec = lax.fori_loop(0, _NCH, p1c, jnp.full((16,), _NEG, jnp.float32))
  row_max = jnp.max(mvec)

  def p2(u, c):
    base = u * (_K2 * 16)
    acc = sm[pl.ds(base, 16)]
    for k in range(1, _K2):
      acc = jnp.maximum(acc, sm[pl.ds(base + k * 16, 16)])
    sm2s[pl.ds(u * 16, 16)] = jnp.sort(acc)
    return c

  lax.fori_loop(0, _S2, p2, 0)

  gidx = jnp.where(iota < _S2, iota * 16 + 11, 0)
  g = plsc.load_gather(sm2s, [gidx], mask=iota < _S2)
  theta = jnp.min(jnp.where(iota < _S2, g, jnp.float32(1e30)))

  def p4(s, cur):
    v = sm[pl.ds(s * 16, 16)]
    m = v >= theta
    plsc.store_compressed(hotidx.at[pl.ds(cur, 16)], s * 16 + iota, mask=m)
    return cur + plsc.all_reduce_population_count(m)[0]

  nhot = lax.fori_loop(0, _S1, p4, 0)

  for j in range(_NCV + 2):
    cand[pl.ds(j * 16, 16)] = jnp.full((16,), _NEG, jnp.float32)

  def p5(j, ccur):
    e = hotidx[pl.ds(j, 16)][0]
    base = lax.shift_right_arithmetic(e, 4) * 400 + (e & 15)
    g1 = plsc.load_gather(row, [base + iota * 16])
    m1 = g1 >= theta
    plsc.store_compressed(cand.at[pl.ds(ccur, 16)], g1, mask=m1)
    ccur = jnp.minimum(ccur + plsc.all_reduce_population_count(m1)[0], _CAP)
    mk = iota < (_K1 - 16)
    g2 = plsc.load_gather(row, [base + (16 + jnp.where(mk, iota, 0)) * 16],
                          mask=mk)
    m2 = (g2 >= theta) & mk
    plsc.store_compressed(cand.at[pl.ds(ccur, 16)], g2, mask=m2)
    return jnp.minimum(ccur + plsc.all_reduce_population_count(m2)[0], _CAP)

  lax.fori_loop(0, nhot, p5, 0)

  vs = [jnp.sort(cand[pl.ds(j * 16, 16)]) for j in range(_NCV)]
  width = 1
  while width < _NCV:
    for j in range(_NCV):
      stage[pl.ds(j * 16, 16)] = vs[j]
    vs = [stage[pl.ds(j * 16, 16)] for j in range(_NCV)]
    runs = []
    for i in range(0, _NCV, 2 * width):
      runs.extend(_merge(vs[i:i + width], vs[i + width:i + 2 * width]))
    vs = runs
    width *= 2
  s_asc = vs
  d = [lax.rev(s_asc[_NCV - 1 - j], (0,)) for j in range(_NCV)]

  kth = s_asc[(_CAP - _TOP_K) // 16][(_CAP - _TOP_K) % 16]

  es, cs = [], []
  carry = jnp.float32(0.0)
  for j in range(_NCV):
    e = jnp.where(d[j] >= kth, jnp.exp((d[j] - row_max) * _INV_T), 0.0)
    c = plsc.cumsum(e) + carry
    carry = c[15]
    es.append(e)
    cs.append(c)
  z = carry

  t_acc = jnp.full((16,), jnp.float32(1e30))
  zf_acc = jnp.zeros((16,), jnp.float32)
  for j in range(_NCV):
    keep = (d[j] >= kth) & ((cs[j] - es[j]) <= _TOP_P * z)
    t_acc = jnp.minimum(t_acc, jnp.where(keep, d[j], jnp.float32(1e30)))
    zf_acc = zf_acc + jnp.where(keep, es[j], 0.0)
  t_thr = jnp.min(t_acc)
  z_final = jnp.sum(zf_acc)

  lse = _log(jnp.full((16,), z_final))[0] + row_max * _INV_T
  fill = jnp.float32(_FILTER_VALUE) - lse

  def p7c(c, cc):
    def p7(s, c2):
      base = c * _CHE + s * (_K1 * 16)
      for k in range(_K1):
        v = row[pl.ds(base + k * 16, 16)]
        row[pl.ds(base + k * 16, 16)] = jnp.where(v >= t_thr,
                                                  v * _INV_T - lse, fill)
      return c2

    lax.fori_loop(0, _GPC, p7, 0)
    off = c * _CHE
    pltpu.async_copy(row.at[pl.ds(off, _CHE)], out_row.at[pl.ds(off, _CHE)],
                     sem_out)
    return cc

  lax.fori_loop(0, _NCH, p7c, 0)
  pltpu.make_async_copy(row, out_row, sem_out).wait()


@functools.cache
def _build():
  info = plsc.get_sparse_core_info()
  nc, ns = info.num_cores, info.num_subcores
  rows_per = _N_ROWS // (nc * ns)
  mesh = plsc.VectorSubcoreMesh(core_axis_name="c", subcore_axis_name="s")

  def body(x_hbm, out_hbm, row, sm, sm2s, hotidx, cand, stage, sem_in,
           sem_out):
    wid = lax.axis_index("s") * nc + lax.axis_index("c")

    def per_row(r, c):
      _row_body(wid * rows_per + r, x_hbm, out_hbm, row, sm, sm2s, hotidx,
                cand, stage, sem_in, sem_out)
      return c

    lax.fori_loop(0, rows_per, per_row, 0)

  return pl.kernel(
      body,
      out_type=jax.ShapeDtypeStruct((_N_ROWS, _N_COLS), jnp.float32),
      mesh=mesh,
      compiler_params=pltpu.CompilerParams(needs_layout_passes=False,
                                           use_tc_tiling_on_sc=False),
      scratch_types=[
          pltpu.VMEM((_N_COLS,), jnp.float32),
          pltpu.VMEM((_S1 * 16,), jnp.float32),
          pltpu.VMEM((_S2 * 16,), jnp.float32),
          pltpu.VMEM((_S1 * 16 + 16,), jnp.int32),
          pltpu.VMEM(((_NCV + 2) * 16,), jnp.float32),
          pltpu.VMEM((_CAP,), jnp.float32),
          pltpu.SemaphoreType.DMA,
          pltpu.SemaphoreType.DMA,
      ],
  )


def kernel(logits):
  return _build()(logits)


# speedup vs baseline: 1.3406x; 1.3406x over previous
import functools

import jax
import jax.numpy as jnp
from jax import lax
from jax.experimental import pallas as pl
from jax.experimental.pallas import tpu as pltpu
from jax.experimental.pallas import tpu_sc as plsc

_TEMPERATURE = 0.8
_INV_T = 1.0 / _TEMPERATURE
_TOP_K = 50
_TOP_P = 0.95
_FILTER_VALUE = -1e9

_N_ROWS = 128
_N_COLS = 100000
_NV = _N_COLS // 16
_K1 = 25
_S1 = _NV // _K1
_K2 = 25
_S2 = _S1 // _K2
_CAP = 512
_NCV = _CAP // 16
_NEG = -1e30
_LN2 = 0.6931471805599453


def _bitonic_fix(vs):
  if len(vs) == 1:
    return [jnp.sort(vs[0])]
  h = len(vs) // 2
  lo = [jnp.minimum(vs[j], vs[j + h]) for j in range(h)]
  hi = [jnp.maximum(vs[j], vs[j + h]) for j in range(h)]
  return _bitonic_fix(lo) + _bitonic_fix(hi)


def _merge(a, b):
  rb = [lax.rev(x, (0,)) for x in reversed(b)]
  lo = [jnp.minimum(x, y) for x, y in zip(a, rb)]
  hi = [jnp.maximum(x, y) for x, y in zip(a, rb)]
  return _bitonic_fix(lo) + _bitonic_fix(hi)


def _log(x):
  b = plsc.bitcast(x, jnp.int32)
  e2 = (lax.shift_right_arithmetic(b, 23) - 127).astype(jnp.float32)
  m = plsc.bitcast((b & 0x007FFFFF) | 0x3F800000, jnp.float32)
  t = (m - 1.0) / (m + 1.0)
  t2 = t * t
  p = 1.0 + t2 * (1 / 3 + t2 * (1 / 5 + t2 * (1 / 7 + t2 * (1 / 9))))
  return e2 * _LN2 + 2.0 * t * p


def _row_body(row_id, x_hbm, out_hbm, row, sm, sm2s, hotidx, cand, stage):
  iota = lax.iota(jnp.int32, 16)
  pltpu.sync_copy(x_hbm.at[row_id], row)

  def p1(s, mcarry):
    base = s * (_K1 * 16)
    acc = row[pl.ds(base, 16)]
    for k in range(1, _K1):
      acc = jnp.maximum(acc, row[pl.ds(base + k * 16, 16)])
    sm[pl.ds(s * 16, 16)] = acc
    return jnp.maximum(mcarry, acc)

  mvec = lax.fori_loop(0, _S1, p1, jnp.full((16,), _NEG, jnp.float32))
  row_max = jnp.max(mvec)

  def p2(u, c):
    base = u * (_K2 * 16)
    acc = sm[pl.ds(base, 16)]
    for k in range(1, _K2):
      acc = jnp.maximum(acc, sm[pl.ds(base + k * 16, 16)])
    sm2s[pl.ds(u * 16, 16)] = jnp.sort(acc)
    return c

  lax.fori_loop(0, _S2, p2, 0)

  gidx = jnp.where(iota < _S2, iota * 16 + 11, 0)
  g = plsc.load_gather(sm2s, [gidx], mask=iota < _S2)
  theta = jnp.min(jnp.where(iota < _S2, g, jnp.float32(1e30)))

  def p4(s, cur):
    v = sm[pl.ds(s * 16, 16)]
    m = v >= theta
    plsc.store_compressed(hotidx.at[pl.ds(cur, 16)], s * 16 + iota, mask=m)
    return cur + plsc.all_reduce_population_count(m)[0]

  nhot = lax.fori_loop(0, _S1, p4, 0)

  for j in range(_NCV + 2):
    cand[pl.ds(j * 16, 16)] = jnp.full((16,), _NEG, jnp.float32)

  def p5(j, ccur):
    e = hotidx[pl.ds(j, 16)][0]
    base = lax.shift_right_arithmetic(e, 4) * 400 + (e & 15)
    g1 = plsc.load_gather(row, [base + iota * 16])
    m1 = g1 >= theta
    plsc.store_compressed(cand.at[pl.ds(ccur, 16)], g1, mask=m1)
    ccur = jnp.minimum(ccur + plsc.all_reduce_population_count(m1)[0], _CAP)
    mk = iota < (_K1 - 16)
    g2 = plsc.load_gather(row, [base + (16 + jnp.where(mk, iota, 0)) * 16],
                          mask=mk)
    m2 = (g2 >= theta) & mk
    plsc.store_compressed(cand.at[pl.ds(ccur, 16)], g2, mask=m2)
    return jnp.minimum(ccur + plsc.all_reduce_population_count(m2)[0], _CAP)

  lax.fori_loop(0, nhot, p5, 0)

  vs = [jnp.sort(cand[pl.ds(j * 16, 16)]) for j in range(_NCV)]
  width = 1
  while width < _NCV:
    for j in range(_NCV):
      stage[pl.ds(j * 16, 16)] = vs[j]
    vs = [stage[pl.ds(j * 16, 16)] for j in range(_NCV)]
    runs = []
    for i in range(0, _NCV, 2 * width):
      runs.extend(_merge(vs[i:i + width], vs[i + width:i + 2 * width]))
    vs = runs
    width *= 2
  s_asc = vs
  d = [lax.rev(s_asc[_NCV - 1 - j], (0,)) for j in range(_NCV)]

  kth = s_asc[(_CAP - _TOP_K) // 16][(_CAP - _TOP_K) % 16]

  es, cs = [], []
  carry = jnp.float32(0.0)
  for j in range(_NCV):
    e = jnp.where(d[j] >= kth, jnp.exp((d[j] - row_max) * _INV_T), 0.0)
    c = plsc.cumsum(e) + carry
    carry = c[15]
    es.append(e)
    cs.append(c)
  z = carry

  t_acc = jnp.full((16,), jnp.float32(1e30))
  zf_acc = jnp.zeros((16,), jnp.float32)
  for j in range(_NCV):
    keep = (d[j] >= kth) & ((cs[j] - es[j]) <= _TOP_P * z)
    t_acc = jnp.minimum(t_acc, jnp.where(keep, d[j], jnp.float32(1e30)))
    zf_acc = zf_acc + jnp.where(keep, es[j], 0.0)
  t_thr = jnp.min(t_acc)
  z_final = jnp.sum(zf_acc)

  lse = _log(jnp.full((16,), z_final))[0] + row_max * _INV_T
  fill = jnp.float32(_FILTER_VALUE) - lse

  def p7(s, c):
    base = s * (_K1 * 16)
    for k in range(_K1):
      v = row[pl.ds(base + k * 16, 16)]
      row[pl.ds(base + k * 16, 16)] = jnp.where(v >= t_thr,
                                                v * _INV_T - lse, fill)
    return c

  lax.fori_loop(0, _S1, p7, 0)
  pltpu.sync_copy(row, out_hbm.at[row_id])


@functools.cache
def _build():
  info = plsc.get_sparse_core_info()
  nc, ns = info.num_cores, info.num_subcores
  rows_per = _N_ROWS // (nc * ns)
  mesh = plsc.VectorSubcoreMesh(core_axis_name="c", subcore_axis_name="s")

  def body(x_hbm, out_hbm, row, sm, sm2s, hotidx, cand, stage):
    wid = lax.axis_index("s") * nc + lax.axis_index("c")

    def per_row(r, c):
      _row_body(wid * rows_per + r, x_hbm, out_hbm, row, sm, sm2s, hotidx,
                cand, stage)
      return c

    lax.fori_loop(0, rows_per, per_row, 0)

  return pl.kernel(
      body,
      out_type=jax.ShapeDtypeStruct((_N_ROWS, _N_COLS), jnp.float32),
      mesh=mesh,
      compiler_params=pltpu.CompilerParams(needs_layout_passes=False, use_tc_tiling_on_sc=False),
      scratch_types=[
          pltpu.VMEM((_N_COLS,), jnp.float32),
          pltpu.VMEM((_S1 * 16,), jnp.float32),
          pltpu.VMEM((_S2 * 16,), jnp.float32),
          pltpu.VMEM((_S1 * 16 + 16,), jnp.int32),
          pltpu.VMEM(((_NCV + 2) * 16,), jnp.float32),
          pltpu.VMEM((_CAP,), jnp.float32),
      ],
  )


def kernel(logits):
  return _build()(logits)


# speedup vs baseline: 1.8708x; 1.3954x over previous
import functools

import jax
import jax.numpy as jnp
from jax import lax
from jax.experimental import pallas as pl
from jax.experimental.pallas import tpu as pltpu
from jax.experimental.pallas import tpu_sc as plsc

_TEMPERATURE = 0.8
_INV_T = 1.0 / _TEMPERATURE
_TOP_K = 50
_TOP_P = 0.95
_FILTER_VALUE = -1e9

_N_ROWS = 128
_N_COLS = 100000
_NV = _N_COLS // 16
_K1 = 25
_S1 = _NV // _K1
_K2 = 25
_S2 = _S1 // _K2
_CAP = 512
_NCV = _CAP // 16
_NEG = -1e30
_LN2 = 0.6931471805599453


def _bitonic_fix(vs):
  if len(vs) == 1:
    return [jnp.sort(vs[0])]
  h = len(vs) // 2
  lo = [jnp.minimum(vs[j], vs[j + h]) for j in range(h)]
  hi = [jnp.maximum(vs[j], vs[j + h]) for j in range(h)]
  return _bitonic_fix(lo) + _bitonic_fix(hi)


def _merge(a, b):
  rb = [lax.rev(x, (0,)) for x in reversed(b)]
  lo = [jnp.minimum(x, y) for x, y in zip(a, rb)]
  hi = [jnp.maximum(x, y) for x, y in zip(a, rb)]
  return _bitonic_fix(lo) + _bitonic_fix(hi)


def _log(x):
  b = plsc.bitcast(x, jnp.int32)
  e2 = (lax.shift_right_arithmetic(b, 23) - 127).astype(jnp.float32)
  m = plsc.bitcast((b & 0x007FFFFF) | 0x3F800000, jnp.float32)
  t = (m - 1.0) / (m + 1.0)
  t2 = t * t
  p = 1.0 + t2 * (1 / 3 + t2 * (1 / 5 + t2 * (1 / 7 + t2 * (1 / 9))))
  return e2 * _LN2 + 2.0 * t * p


def _row_body(row_id, x_hbm, out_hbm, row, sm, sm2s, hotidx, cand, stage):
  iota = lax.iota(jnp.int32, 16)
  pltpu.sync_copy(x_hbm.at[row_id], row)

  def p1(s, mcarry):
    base = s * (_K1 * 16)
    acc = row[pl.ds(base, 16)]
    for k in range(1, _K1):
      acc = jnp.maximum(acc, row[pl.ds(base + k * 16, 16)])
    sm[pl.ds(s * 16, 16)] = acc
    return jnp.maximum(mcarry, acc)

  mvec = lax.fori_loop(0, _S1, p1, jnp.full((16,), _NEG, jnp.float32))
  row_max = jnp.max(mvec)

  def p2(u, c):
    base = u * (_K2 * 16)
    acc = sm[pl.ds(base, 16)]
    for k in range(1, _K2):
      acc = jnp.maximum(acc, sm[pl.ds(base + k * 16, 16)])
    sm2s[pl.ds(u * 16, 16)] = jnp.sort(acc)
    return c

  lax.fori_loop(0, _S2, p2, 0)

  gidx = jnp.where(iota < _S2, iota * 16 + 11, 0)
  g = plsc.load_gather(sm2s, [gidx], mask=iota < _S2)
  theta = jnp.min(jnp.where(iota < _S2, g, jnp.float32(1e30)))

  def p4(s, cur):
    v = sm[pl.ds(s * 16, 16)]
    m = v >= theta
    plsc.store_compressed(hotidx.at[pl.ds(cur, 16)], s * 16 + iota, mask=m)
    return cur + plsc.all_reduce_population_count(m)[0]

  nhot = lax.fori_loop(0, _S1, p4, 0)

  for j in range(_NCV + 2):
    cand[pl.ds(j * 16, 16)] = jnp.full((16,), _NEG, jnp.float32)

  def p5(j, ccur):
    e = hotidx[pl.ds(j, 16)][0]
    base = lax.shift_right_arithmetic(e, 4) * 400 + (e & 15)
    g1 = plsc.load_gather(row, [base + iota * 16])
    m1 = g1 >= theta
    plsc.store_compressed(cand.at[pl.ds(ccur, 16)], g1, mask=m1)
    ccur = jnp.minimum(ccur + plsc.all_reduce_population_count(m1)[0], _CAP)
    mk = iota < (_K1 - 16)
    g2 = plsc.load_gather(row, [base + (16 + jnp.where(mk, iota, 0)) * 16],
                          mask=mk)
    m2 = (g2 >= theta) & mk
    plsc.store_compressed(cand.at[pl.ds(ccur, 16)], g2, mask=m2)
    return jnp.minimum(ccur + plsc.all_reduce_population_count(m2)[0], _CAP)

  lax.fori_loop(0, nhot, p5, 0)

  vs = [jnp.sort(cand[pl.ds(j * 16, 16)]) for j in range(_NCV)]
  width = 1
  while width < _NCV:
    for j in range(_NCV):
      stage[pl.ds(j * 16, 16)] = vs[j]
    vs = [stage[pl.ds(j * 16, 16)] for j in range(_NCV)]
    runs = []
    for i in range(0, _NCV, 2 * width):
      runs.extend(_merge(vs[i:i + width], vs[i + width:i + 2 * width]))
    vs = runs
    width *= 2
  s_asc = vs
  d = [lax.rev(s_asc[_NCV - 1 - j], (0,)) for j in range(_NCV)]

  kth = s_asc[(_CAP - _TOP_K) // 16][(_CAP - _TOP_K) % 16]

  es, cs = [], []
  carry = jnp.float32(0.0)
  for j in range(_NCV):
    e = jnp.where(d[j] >= kth, jnp.exp((d[j] - row_max) * _INV_T), 0.0)
    c = plsc.cumsum(e) + carry
    carry = c[15]
    es.append(e)
    cs.append(c)
  z = carry

  t_acc = jnp.full((16,), jnp.float32(1e30))
  zf_acc = jnp.zeros((16,), jnp.float32)
  for j in range(_NCV):
    keep = (d[j] >= kth) & ((cs[j] - es[j]) <= _TOP_P * z)
    t_acc = jnp.minimum(t_acc, jnp.where(keep, d[j], jnp.float32(1e30)))
    zf_acc = zf_acc + jnp.where(keep, es[j], 0.0)
  t_thr = jnp.min(t_acc)
  z_final = jnp.sum(zf_acc)

  lse = _log(jnp.full((16,), z_final))[0] + row_max * _INV_T
  fill = jnp.float32(_FILTER_VALUE) - lse

  def p7(s, c):
    base = s * (_K1 * 16)
    for k in range(_K1):
      v = row[pl.ds(base + k * 16, 16)]
      row[pl.ds(base + k * 16, 16)] = jnp.where(v >= t_thr,
                                                v * _INV_T - lse, fill)
    return c

  lax.fori_loop(0, _S1, p7, 0)
  pltpu.sync_copy(row, out_hbm.at[row_id])


@functools.cache
def _build():
  info = plsc.get_sparse_core_info()
  nc, ns = info.num_cores, info.num_subcores
  rows_per = _N_ROWS // (nc * ns)
  mesh = plsc.VectorSubcoreMesh(core_axis_name="c", subcore_axis_name="s")

  def body(x_hbm, out_hbm, row, sm, sm2s, hotidx, cand, stage):
    wid = lax.axis_index("s") * nc + lax.axis_index("c")

    def per_row(r, c):
      _row_body(wid * rows_per + r, x_hbm, out_hbm, row, sm, sm2s, hotidx,
                cand, stage)
      return c

    lax.fori_loop(0, rows_per, per_row, 0)

  return pl.kernel(
      body,
      out_type=jax.ShapeDtypeStruct((_N_ROWS, _N_COLS), jnp.float32),
      mesh=mesh,
      compiler_params=pltpu.CompilerParams(needs_layout_passes=False),
      scratch_types=[
          pltpu.VMEM((_N_COLS,), jnp.float32),
          pltpu.VMEM((_S1 * 16,), jnp.float32),
          pltpu.VMEM((_S2 * 16,), jnp.float32),
          pltpu.VMEM((_S1 * 16 + 16,), jnp.int32),
          pltpu.VMEM(((_NCV + 2) * 16,), jnp.float32),
          pltpu.VMEM((_CAP,), jnp.float32),
      ],
  )


def kernel(logits):
  return _build()(logits)
